# baseline (device time: 87126 ns/iter reference)
import jax
import jax.numpy as jnp
from jax import lax
from jax.experimental import pallas as pl
from jax.experimental.pallas import tpu as pltpu

N_DEV = 32


def kernel(x, Wg, Wu, Wd):
    m, _ = x.shape
    d_out = Wd.shape[1]
    chunk = m // N_DEV

    xb = x.astype(jnp.bfloat16)
    wgb = Wg.astype(jnp.bfloat16)
    wub = Wu.astype(jnp.bfloat16)
    wdb = Wd.astype(jnp.bfloat16)

    def rows(c):
        return pl.ds(c * chunk, chunk)

    def body(x_ref, wg_ref, wu_ref, wd_ref, out_ref,
             part_ref, rs_buf, send_sems, rs_sems, ag_sems):
        d = lax.axis_index("i")

        gate = jnp.dot(x_ref[:], wg_ref[:], preferred_element_type=jnp.float32)
        up = jnp.dot(x_ref[:], wu_ref[:], preferred_element_type=jnp.float32)
        act = (gate * (up * lax.logistic(up))).astype(jnp.bfloat16)
        part = jnp.dot(act, wd_ref[:], preferred_element_type=jnp.float32)
        part_ref[:] = part.astype(jnp.bfloat16)

        bar = pltpu.get_barrier_semaphore()
        for k in range(1, N_DEV):
            tgt = lax.rem(d + k, N_DEV)
            pl.semaphore_signal(bar, inc=1, device_id=(tgt,),
                                device_id_type=pl.DeviceIdType.MESH)
        pl.semaphore_wait(bar, N_DEV - 1)

        rs_sends = []
        for k in range(1, N_DEV):
            tgt = lax.rem(d + k, N_DEV)
            rdma = pltpu.make_async_remote_copy(
                src_ref=part_ref.at[rows(tgt), :],
                dst_ref=rs_buf.at[d],
                send_sem=send_sems.at[k],
                recv_sem=rs_sems.at[d],
                device_id=(tgt,),
                device_id_type=pl.DeviceIdType.MESH,
            )
            rdma.start()
            rs_sends.append(rdma)

        rs_buf[d] = part_ref[rows(d), :]

        for k in range(1, N_DEV):
            src = lax.rem(d + k, N_DEV)
            pltpu.make_async_remote_copy(
                src_ref=rs_buf.at[src],
                dst_ref=rs_buf.at[src],
                send_sem=send_sems.at[k],
                recv_sem=rs_sems.at[src],
                device_id=(d,),
                device_id_type=pl.DeviceIdType.MESH,
            ).wait_recv()

        total = jnp.sum(rs_buf[:].astype(jnp.float32), axis=0)
        out_ref[rows(d), :] = total.astype(jnp.bfloat16)

        ag_sends = []
        for k in range(1, N_DEV):
            tgt = lax.rem(d + k, N_DEV)
            rs_sends[k - 1].wait_send()
            rdma = pltpu.make_async_remote_copy(
                src_ref=out_ref.at[rows(d), :],
                dst_ref=out_ref.at[rows(d), :],
                send_sem=send_sems.at[k],
                recv_sem=ag_sems.at[d],
                device_id=(tgt,),
                device_id_type=pl.DeviceIdType.MESH,
            )
            rdma.start()
            ag_sends.append(rdma)

        for k in range(1, N_DEV):
            src = lax.rem(d + k, N_DEV)
            pltpu.make_async_remote_copy(
                src_ref=out_ref.at[rows(src), :],
                dst_ref=out_ref.at[rows(src), :],
                send_sem=send_sems.at[k],
                recv_sem=ag_sems.at[src],
                device_id=(d,),
                device_id_type=pl.DeviceIdType.MESH,
            ).wait_recv()

        for rdma in ag_sends:
            rdma.wait_send()

    return pl.pallas_call(
        body,
        out_shape=jax.ShapeDtypeStruct((m, d_out), jnp.bfloat16),
        in_specs=[pl.BlockSpec(memory_space=pltpu.VMEM)] * 4,
        out_specs=pl.BlockSpec(memory_space=pltpu.VMEM),
        scratch_shapes=[
            pltpu.VMEM((m, d_out), jnp.bfloat16),
            pltpu.VMEM((N_DEV, chunk, d_out), jnp.bfloat16),
            pltpu.SemaphoreType.DMA((N_DEV,)),
            pltpu.SemaphoreType.DMA((N_DEV,)),
            pltpu.SemaphoreType.DMA((N_DEV,)),
        ],
        compiler_params=pltpu.CompilerParams(collective_id=0),
    )(xb, wgb, wub, wdb)


# device time: 82143 ns/iter; 1.0607x vs baseline; 1.0607x over previous
import jax
import jax.numpy as jnp
from jax import lax
from jax.experimental import pallas as pl
from jax.experimental.pallas import tpu as pltpu

N_DEV = 32
N_WAVES = 4
PER_WAVE = N_DEV // N_WAVES


def kernel(x, Wg, Wu, Wd):
    m, _ = x.shape
    d_out = Wd.shape[1]
    chunk = m // N_DEV

    xb = x.astype(jnp.bfloat16)
    wgb = Wg.astype(jnp.bfloat16)
    wub = Wu.astype(jnp.bfloat16)
    wdb = Wd.astype(jnp.bfloat16)

    def rows(c):
        return pl.ds(c * chunk, chunk)

    def body(x_ref, wg_ref, wu_ref, wd_ref, out_ref,
             part_ref, rs_buf, send_sems, rs_sems, ag_sems):
        d = lax.axis_index("i")

        bar = pltpu.get_barrier_semaphore()
        for k in range(1, N_DEV):
            tgt = lax.rem(d + k, N_DEV)
            pl.semaphore_signal(bar, inc=1, device_id=(tgt,),
                                device_id_type=pl.DeviceIdType.MESH)

        gate = jnp.dot(x_ref[:], wg_ref[:], preferred_element_type=jnp.float32)
        up = jnp.dot(x_ref[:], wu_ref[:], preferred_element_type=jnp.float32)
        act = (gate * (up * lax.logistic(up))).astype(jnp.bfloat16)

        pl.semaphore_wait(bar, N_DEV - 1)

        def rs_send(j):
            return pltpu.make_async_remote_copy(
                src_ref=part_ref.at[rows(j), :],
                dst_ref=rs_buf.at[d],
                send_sem=send_sems.at[j],
                recv_sem=rs_sems.at[d],
                device_id=(j,),
                device_id_type=pl.DeviceIdType.MESH,
            )

        for b in range(N_WAVES):
            rb = slice(b * PER_WAVE * chunk, (b + 1) * PER_WAVE * chunk)
            part_b = jnp.dot(act[rb], wd_ref[:],
                             preferred_element_type=jnp.float32)
            part_ref[rb, :] = part_b.astype(jnp.bfloat16)
            for j in range(b * PER_WAVE, (b + 1) * PER_WAVE):
                @pl.when(j != d)
                def _(j=j):
                    rs_send(j).start()

                @pl.when(j == d)
                def _(j=j):
                    rs_buf[j] = part_ref[rows(j), :]

        for j in range(N_DEV):
            @pl.when(j != d)
            def _(j=j):
                pltpu.make_async_remote_copy(
                    src_ref=rs_buf.at[j],
                    dst_ref=rs_buf.at[j],
                    send_sem=send_sems.at[j],
                    recv_sem=rs_sems.at[j],
                    device_id=(d,),
                    device_id_type=pl.DeviceIdType.MESH,
                ).wait_recv()

        total = jnp.sum(rs_buf[:].astype(jnp.float32), axis=0)
        rows_d = pl.ds(d * chunk, chunk)
        out_ref[rows_d, :] = total.astype(jnp.bfloat16)

        def ag_send(j):
            return pltpu.make_async_remote_copy(
                src_ref=out_ref.at[rows_d, :],
                dst_ref=out_ref.at[rows_d, :],
                send_sem=send_sems.at[j],
                recv_sem=ag_sems.at[d],
                device_id=(j,),
                device_id_type=pl.DeviceIdType.MESH,
            )

        for j in range(N_DEV):
            @pl.when(j != d)
            def _(j=j):
                rs_send(j).wait_send()
                ag_send(j).start()

        for j in range(N_DEV):
            @pl.when(j != d)
            def _(j=j):
                pltpu.make_async_remote_copy(
                    src_ref=out_ref.at[rows(j), :],
                    dst_ref=out_ref.at[rows(j), :],
                    send_sem=send_sems.at[j],
                    recv_sem=ag_sems.at[j],
                    device_id=(d,),
                    device_id_type=pl.DeviceIdType.MESH,
                ).wait_recv()

        for j in range(N_DEV):
            @pl.when(j != d)
            def _(j=j):
                ag_send(j).wait_send()

    return pl.pallas_call(
        body,
        out_shape=jax.ShapeDtypeStruct((m, d_out), jnp.bfloat16),
        in_specs=[pl.BlockSpec(memory_space=pltpu.VMEM)] * 4,
        out_specs=pl.BlockSpec(memory_space=pltpu.VMEM),
        scratch_shapes=[
            pltpu.VMEM((m, d_out), jnp.bfloat16),
            pltpu.VMEM((N_DEV, chunk, d_out), jnp.bfloat16),
            pltpu.SemaphoreType.DMA((N_DEV,)),
            pltpu.SemaphoreType.DMA((N_DEV,)),
            pltpu.SemaphoreType.DMA((N_DEV,)),
        ],
        compiler_params=pltpu.CompilerParams(collective_id=0),
    )(xb, wgb, wub, wdb)
